# baseline (speedup 1.0000x reference)
import functools

import jax
import jax.numpy as jnp
from jax import lax
from jax.experimental import pallas as pl
from jax.experimental.pallas import tpu as pltpu
from jax.experimental.pallas import tpu_sc as plsc

NC = 2
NS = 16
NW = NC * NS
CHUNK = 128


def _mesh():
    return plsc.VectorSubcoreMesh(
        core_axis_name="c", subcore_axis_name="s",
        num_cores=NC, num_subcores=NS)


@functools.lru_cache(maxsize=None)
def _make_deg_kernel(npn: int, k: int):
    rows_per_tile = npn // NS

    @functools.partial(
        pl.kernel, mesh=_mesh(),
        out_type=jax.ShapeDtypeStruct((NC, 2, npn), jnp.float32),
        scratch_types=[
            pltpu.VMEM((k, CHUNK), jnp.int32),
            pltpu.VMEM((k, CHUNK), jnp.int32),
            pltpu.VMEM((CHUNK,), jnp.float32),
            pltpu.VMEM((CHUNK,), jnp.float32),
            pltpu.VMEM_SHARED((npn,), jnp.float32),
            pltpu.VMEM_SHARED((npn,), jnp.float32),
        ],
    )
    def deg_kernel(src_hbm, dst_hbm, out_hbm,
                   src_v, dst_v, ones_v, zeros_v, dego_s, degi_s):
        cid = lax.axis_index("c")
        sid = lax.axis_index("s")
        wid = sid * NC + cid

        def fill(i, _):
            ones_v[pl.ds(i * 16, 16)] = jnp.full((16,), 1.0, jnp.float32)
            zeros_v[pl.ds(i * 16, 16)] = jnp.zeros((16,), jnp.float32)
            return 0
        lax.fori_loop(0, CHUNK // 16, fill, 0)

        base = sid * rows_per_tile

        def zrow(i, _):
            pltpu.sync_copy(zeros_v, dego_s.at[pl.ds(base + i * CHUNK, CHUNK)])
            pltpu.sync_copy(zeros_v, degi_s.at[pl.ds(base + i * CHUNK, CHUNK)])
            return 0
        lax.fori_loop(0, rows_per_tile // CHUNK, zrow, 0)
        plsc.subcore_barrier()

        pltpu.sync_copy(src_hbm.at[wid], src_v)
        pltpu.sync_copy(dst_hbm.at[wid], dst_v)

        def body(j, _):
            pltpu.sync_copy(ones_v, dego_s.at[src_v.at[j]], add=True)
            pltpu.sync_copy(ones_v, degi_s.at[dst_v.at[j]], add=True)
            return 0
        lax.fori_loop(0, k, body, 0)
        plsc.subcore_barrier()

        pltpu.sync_copy(dego_s.at[pl.ds(base, rows_per_tile)],
                        out_hbm.at[cid, 0, pl.ds(base, rows_per_tile)])
        pltpu.sync_copy(degi_s.at[pl.ds(base, rows_per_tile)],
                        out_hbm.at[cid, 1, pl.ds(base, rows_per_tile)])

    return deg_kernel


@functools.lru_cache(maxsize=None)
def _make_agg_kernel(npn: int, d: int, k: int):
    rows_per_tile = npn // NS

    @functools.partial(
        pl.kernel, mesh=_mesh(),
        out_type=jax.ShapeDtypeStruct((NC, npn, d), jnp.float32),
        scratch_types=[
            pltpu.VMEM((CHUNK,), jnp.int32),
            pltpu.VMEM((CHUNK,), jnp.int32),
            pltpu.VMEM((CHUNK,), jnp.int32),
            pltpu.VMEM((CHUNK,), jnp.int32),
            pltpu.VMEM((CHUNK, d), jnp.float32),
            pltpu.VMEM((CHUNK, d), jnp.float32),
            pltpu.VMEM_SHARED((npn, d), jnp.float32),
            pltpu.SemaphoreType.DMA,
            pltpu.SemaphoreType.DMA,
            pltpu.SemaphoreType.DMA,
            pltpu.SemaphoreType.DMA,
            pltpu.SemaphoreType.DMA,
            pltpu.SemaphoreType.DMA,
        ],
    )
    def agg_kernel(g_hbm, src_hbm, dst_hbm, out_hbm,
                   src_a, src_b, dst_a, dst_b, rows_a, rows_b, agg_s,
                   sem_isa, sem_isb, sem_ida, sem_idb, sem_ga, sem_gb):
        cid = lax.axis_index("c")
        sid = lax.axis_index("s")
        wid = sid * NC + cid

        nv = CHUNK * d // 16

        def fz(i, _):
            rows_a[i // (d // 16), pl.ds((i % (d // 16)) * 16, 16)] = (
                jnp.zeros((16,), jnp.float32))
            return 0
        lax.fori_loop(0, nv, fz, 0)

        base = sid * rows_per_tile

        def zrow(i, _):
            pltpu.sync_copy(rows_a, agg_s.at[pl.ds(base + i * CHUNK, CHUNK)])
            return 0
        lax.fori_loop(0, rows_per_tile // CHUNK, zrow, 0)
        plsc.subcore_barrier()

        pltpu.async_copy(src_hbm.at[wid, 0], src_a, sem_isa)
        pltpu.async_copy(dst_hbm.at[wid, 0], dst_a, sem_ida)
        pltpu.async_copy(src_hbm.at[wid, 1], src_b, sem_isb)
        pltpu.async_copy(dst_hbm.at[wid, 1], dst_b, sem_idb)
        pltpu.make_async_copy(src_hbm.at[wid, 0], src_a, sem_isa).wait()
        pltpu.async_copy(g_hbm.at[src_a], rows_a, sem_ga)

        def body(i, _):
            j = 2 * i
            more = j + 2 < k

            pltpu.make_async_copy(
                src_hbm.at[wid, j + 1], src_b, sem_isb).wait()
            pltpu.async_copy(g_hbm.at[src_b], rows_b, sem_gb)
            pltpu.make_async_copy(g_hbm.at[src_a], rows_a, sem_ga).wait()

            @pl.when(more)
            def _():
                pltpu.async_copy(src_hbm.at[wid, j + 2], src_a, sem_isa)
            pltpu.make_async_copy(
                dst_hbm.at[wid, j], dst_a, sem_ida).wait()
            pltpu.sync_copy(rows_a, agg_s.at[dst_a], add=True)

            @pl.when(more)
            def _():
                pltpu.async_copy(dst_hbm.at[wid, j + 2], dst_a, sem_ida)
                pltpu.make_async_copy(
                    src_hbm.at[wid, j + 2], src_a, sem_isa).wait()
                pltpu.async_copy(g_hbm.at[src_a], rows_a, sem_ga)
            pltpu.make_async_copy(g_hbm.at[src_b], rows_b, sem_gb).wait()

            @pl.when(more)
            def _():
                pltpu.async_copy(src_hbm.at[wid, j + 3], src_b, sem_isb)
            pltpu.make_async_copy(
                dst_hbm.at[wid, j + 1], dst_b, sem_idb).wait()
            pltpu.sync_copy(rows_b, agg_s.at[dst_b], add=True)

            @pl.when(more)
            def _():
                pltpu.async_copy(dst_hbm.at[wid, j + 3], dst_b, sem_idb)
            return 0
        lax.fori_loop(0, k // 2, body, 0)
        plsc.subcore_barrier()

        pltpu.sync_copy(agg_s.at[pl.ds(base, rows_per_tile)],
                        out_hbm.at[cid, pl.ds(base, rows_per_tile)])

    return agg_kernel


def _norm_cols(degs):
    norm_out = lax.rsqrt(jnp.maximum(degs[:, 0:1] + degs[:, 1:2], 1.0))
    norm_in = lax.rsqrt(jnp.maximum(degs[:, 2:3] + degs[:, 3:4], 1.0))
    return norm_out, norm_in


def _tc_first_body(degs_ref, x_ref, w_ref, g_ref):
    norm_out, _ = _norm_cols(degs_ref[...])
    xw = jnp.dot(x_ref[...], w_ref[...], preferred_element_type=jnp.float32)
    g_ref[...] = xw * norm_out


def _tc_mid_body(degs_ref, agg_ref, b_ref, w_ref, g_ref):
    norm_out, norm_in = _norm_cols(degs_ref[...])
    agg = agg_ref[0] + agg_ref[1]
    h = jnp.maximum(agg * norm_in + b_ref[...][None, :], 0.0)
    hw = jnp.dot(h, w_ref[...], preferred_element_type=jnp.float32)
    g_ref[...] = hw * norm_out


def _tc_last_body(degs_ref, agg_ref, b_ref, out_ref):
    _, norm_in = _norm_cols(degs_ref[...])
    agg = agg_ref[0] + agg_ref[1]
    out_ref[...] = agg * norm_in + b_ref[...][None, :]


def _tc_call(body, out_shape, *args):
    return pl.pallas_call(
        body, out_shape=jax.ShapeDtypeStruct(out_shape, jnp.float32))(*args)


def kernel(features, edge_index, W1, b1, W2, b2, W3, b3):
    n, d_in = features.shape
    e = edge_index.shape[1]
    d_h = W1.shape[1]
    d_out = W3.shape[1]

    k = -(-e // (NW * CHUNK))
    k += k % 2
    ep = NW * k * CHUNK
    npn = -(-(n + 1) // (NS * CHUNK)) * (NS * CHUNK)

    src = edge_index[0]
    dst = edge_index[1]
    pad = ep - e
    src_g = jnp.pad(src, (0, pad)).reshape(NW, k, CHUNK)
    dst_s = jnp.pad(dst, (0, pad), constant_values=n).reshape(NW, k, CHUNK)
    src_d = jnp.pad(src, (0, pad), constant_values=n).reshape(NW, k, CHUNK)

    x_p = jnp.pad(features, ((0, npn - n), (0, 0)))

    deg_parts = _make_deg_kernel(npn, k)(src_d, dst_s)
    degs = jnp.transpose(deg_parts, (2, 1, 0)).reshape(npn, 4)

    w3_p = jnp.pad(W3, ((0, 0), (0, d_h - d_out)))
    b3_p = jnp.pad(b3, (0, d_h - d_out))

    agg = _make_agg_kernel(npn, d_h, k)
    g1 = _tc_call(_tc_first_body, (npn, d_h), degs, x_p, W1)
    a1 = agg(g1, src_g, dst_s)
    g2 = _tc_call(_tc_mid_body, (npn, d_h), degs, a1, b1, W2)
    a2 = agg(g2, src_g, dst_s)
    g3 = _tc_call(_tc_mid_body, (npn, d_h), degs, a2, b2, w3_p)
    a3 = agg(g3, src_g, dst_s)
    logits = _tc_call(_tc_last_body, (npn, d_h), degs, a3, b3_p)
    return logits[:n, :d_out]


# speedup vs baseline: 1.0014x; 1.0014x over previous
import functools

import jax
import jax.numpy as jnp
from jax import lax
from jax.experimental import pallas as pl
from jax.experimental.pallas import tpu as pltpu
from jax.experimental.pallas import tpu_sc as plsc

NC = 2
NS = 16
NW = NC * NS
CHUNK = 128
BLK = 8


def _mesh():
    return plsc.VectorSubcoreMesh(
        core_axis_name="c", subcore_axis_name="s",
        num_cores=NC, num_subcores=NS)


@functools.lru_cache(maxsize=None)
def _make_deg_kernel(npn: int, k: int):
    rows_per_tile = npn // NS

    @functools.partial(
        pl.kernel, mesh=_mesh(),
        out_type=jax.ShapeDtypeStruct((NC, 2, npn), jnp.float32),
        scratch_types=[
            pltpu.VMEM((k, CHUNK), jnp.int32),
            pltpu.VMEM((k, CHUNK), jnp.int32),
            pltpu.VMEM((CHUNK,), jnp.float32),
            pltpu.VMEM((CHUNK,), jnp.float32),
            pltpu.VMEM_SHARED((npn,), jnp.float32),
            pltpu.VMEM_SHARED((npn,), jnp.float32),
        ],
    )
    def deg_kernel(src_hbm, dst_hbm, out_hbm,
                   src_v, dst_v, ones_v, zeros_v, dego_s, degi_s):
        cid = lax.axis_index("c")
        sid = lax.axis_index("s")
        wid = sid * NC + cid

        def fill(i, _):
            ones_v[pl.ds(i * 16, 16)] = jnp.full((16,), 1.0, jnp.float32)
            zeros_v[pl.ds(i * 16, 16)] = jnp.zeros((16,), jnp.float32)
            return 0
        lax.fori_loop(0, CHUNK // 16, fill, 0)

        base = sid * rows_per_tile

        def zrow(i, _):
            pltpu.sync_copy(zeros_v, dego_s.at[pl.ds(base + i * CHUNK, CHUNK)])
            pltpu.sync_copy(zeros_v, degi_s.at[pl.ds(base + i * CHUNK, CHUNK)])
            return 0
        lax.fori_loop(0, rows_per_tile // CHUNK, zrow, 0)
        plsc.subcore_barrier()

        pltpu.sync_copy(src_hbm.at[wid], src_v)
        pltpu.sync_copy(dst_hbm.at[wid], dst_v)

        def body(j, _):
            pltpu.sync_copy(ones_v, dego_s.at[src_v.at[j]], add=True)
            pltpu.sync_copy(ones_v, degi_s.at[dst_v.at[j]], add=True)
            return 0
        lax.fori_loop(0, k, body, 0)
        plsc.subcore_barrier()

        pltpu.sync_copy(dego_s.at[pl.ds(base, rows_per_tile)],
                        out_hbm.at[cid, 0, pl.ds(base, rows_per_tile)])
        pltpu.sync_copy(degi_s.at[pl.ds(base, rows_per_tile)],
                        out_hbm.at[cid, 1, pl.ds(base, rows_per_tile)])

    return deg_kernel


@functools.lru_cache(maxsize=None)
def _make_agg_kernel(npn: int, d: int, k: int):
    rows_per_tile = npn // NS

    nb = k // BLK
    nb2 = nb // 2
    if True:
        @functools.partial(
            pl.kernel, mesh=_mesh(),
            out_type=jax.ShapeDtypeStruct((NC, npn, d), jnp.float32),
            scratch_types=[
                pltpu.VMEM((k, CHUNK), jnp.int32),
                pltpu.VMEM((BLK, CHUNK), jnp.int32),
                pltpu.VMEM((BLK, CHUNK), jnp.int32),
                pltpu.VMEM((CHUNK, d), jnp.float32),
                pltpu.VMEM((CHUNK, d), jnp.float32),
                pltpu.VMEM_SHARED((npn, d), jnp.float32),
                pltpu.SemaphoreType.DMA,
                pltpu.SemaphoreType.DMA,
                pltpu.SemaphoreType.DMA,
                pltpu.SemaphoreType.DMA,
            ],
        )
        def agg_kernel(g_hbm, src_hbm, dst_hbm, out_hbm,
                       dst_v, sblk_a, sblk_b, rows_a, rows_b, agg_s,
                       sem_sa, sem_sb, sem_ga, sem_gb):
            cid = lax.axis_index("c")
            sid = lax.axis_index("s")
            wid = sid * NC + cid

            nv = CHUNK * d // 16

            def fz(i, _):
                rows_a[i // (d // 16), pl.ds((i % (d // 16)) * 16, 16)] = (
                    jnp.zeros((16,), jnp.float32))
                return 0
            lax.fori_loop(0, nv, fz, 0)

            base = sid * rows_per_tile

            def zrow(i, _):
                pltpu.sync_copy(rows_a,
                                agg_s.at[pl.ds(base + i * CHUNK, CHUNK)])
                return 0
            lax.fori_loop(0, rows_per_tile // CHUNK, zrow, 0)
            plsc.subcore_barrier()

            rows = (rows_a, rows_b)
            sem_g = (sem_ga, sem_gb)
            sblk = (sblk_a, sblk_b)
            sem_s = (sem_sa, sem_sb)

            pltpu.sync_copy(src_hbm.at[wid, 0], sblk_a)
            pltpu.async_copy(g_hbm.at[sblk_a.at[0]], rows_a, sem_ga)
            pltpu.async_copy(src_hbm.at[wid, 1], sblk_b, sem_sb)
            pltpu.sync_copy(dst_hbm.at[wid], dst_v)

            def body(m, _):
                c0 = (2 * BLK) * m
                for t in range(2 * BLK):
                    c = c0 + t
                    bsel = (t // BLK) & 1
                    idx_c = sblk[bsel].at[t % BLK]
                    last_t = t == 2 * BLK - 1
                    if t == BLK - 1:
                        pltpu.make_async_copy(
                            src_hbm.at[wid, 2 * m + 1], sblk_b, sem_sb).wait()
                    if not last_t:
                        idx_n = sblk[((t + 1) // BLK) & 1].at[(t + 1) % BLK]
                        pltpu.async_copy(
                            g_hbm.at[idx_n], rows[(t + 1) & 1],
                            sem_g[(t + 1) & 1])
                    else:
                        @pl.when(m + 1 < nb2)
                        def _():
                            pltpu.make_async_copy(
                                src_hbm.at[wid, 2 * m + 2], sblk_a,
                                sem_sa).wait()
                            pltpu.async_copy(
                                g_hbm.at[sblk_a.at[0]], rows[0], sem_g[0])
                    pltpu.make_async_copy(
                        g_hbm.at[idx_c], rows[t & 1], sem_g[t & 1]).wait()
                    if t == BLK - 1:
                        @pl.when(m + 1 < nb2)
                        def _():
                            pltpu.async_copy(
                                src_hbm.at[wid, 2 * m + 2], sblk_a, sem_sa)
                    if last_t:
                        @pl.when(m + 1 < nb2)
                        def _():
                            pltpu.async_copy(
                                src_hbm.at[wid, 2 * m + 3], sblk_b, sem_sb)
                    pltpu.sync_copy(rows[t & 1], agg_s.at[dst_v.at[c]],
                                    add=True)
                return 0
            lax.fori_loop(0, nb2, body, 0)
            plsc.subcore_barrier()

            pltpu.sync_copy(agg_s.at[pl.ds(base, rows_per_tile)],
                            out_hbm.at[cid, pl.ds(base, rows_per_tile)])

    return agg_kernel


def _norm_cols(degs):
    norm_out = lax.rsqrt(jnp.maximum(degs[:, 0:1] + degs[:, 1:2], 1.0))
    norm_in = lax.rsqrt(jnp.maximum(degs[:, 2:3] + degs[:, 3:4], 1.0))
    return norm_out, norm_in


def _tc_first_body(degs_ref, x_ref, w_ref, g_ref):
    norm_out, _ = _norm_cols(degs_ref[...])
    xw = jnp.dot(x_ref[...], w_ref[...], preferred_element_type=jnp.float32)
    g_ref[...] = xw * norm_out


def _tc_mid_body(degs_ref, agg_ref, b_ref, w_ref, g_ref):
    norm_out, norm_in = _norm_cols(degs_ref[...])
    agg = agg_ref[0] + agg_ref[1]
    h = jnp.maximum(agg * norm_in + b_ref[...][None, :], 0.0)
    hw = jnp.dot(h, w_ref[...], preferred_element_type=jnp.float32)
    g_ref[...] = hw * norm_out


def _tc_last_body(degs_ref, agg_ref, b_ref, out_ref):
    _, norm_in = _norm_cols(degs_ref[...])
    agg = agg_ref[0] + agg_ref[1]
    out_ref[...] = agg * norm_in + b_ref[...][None, :]


def _tc_call(body, out_shape, *args):
    return pl.pallas_call(
        body, out_shape=jax.ShapeDtypeStruct(out_shape, jnp.float32))(*args)


def kernel(features, edge_index, W1, b1, W2, b2, W3, b3):
    n, d_in = features.shape
    e = edge_index.shape[1]
    d_h = W1.shape[1]
    d_out = W3.shape[1]

    k = -(-e // (NW * CHUNK))
    k += -k % (2 * BLK)
    ep = NW * k * CHUNK
    npn = -(-(n + 1) // (NS * CHUNK)) * (NS * CHUNK)

    src = edge_index[0]
    dst = edge_index[1]
    pad = ep - e
    src_g = jnp.pad(src, (0, pad)).reshape(NW, k // BLK, BLK, CHUNK)
    dst_s = jnp.pad(dst, (0, pad), constant_values=n).reshape(NW, k, CHUNK)
    src_d = jnp.pad(src, (0, pad), constant_values=n).reshape(NW, k, CHUNK)

    x_p = jnp.pad(features, ((0, npn - n), (0, 0)))

    deg_parts = _make_deg_kernel(npn, k)(src_d, dst_s)
    degs = jnp.transpose(deg_parts, (2, 1, 0)).reshape(npn, 4)

    w3_p = jnp.pad(W3, ((0, 0), (0, d_h - d_out)))
    b3_p = jnp.pad(b3, (0, d_h - d_out))

    agg = _make_agg_kernel(npn, d_h, k)
    g1 = _tc_call(_tc_first_body, (npn, d_h), degs, x_p, W1)
    a1 = agg(g1, src_g, dst_s)
    g2 = _tc_call(_tc_mid_body, (npn, d_h), degs, a1, b1, W2)
    a2 = agg(g2, src_g, dst_s)
    g3 = _tc_call(_tc_mid_body, (npn, d_h), degs, a2, b2, w3_p)
    a3 = agg(g3, src_g, dst_s)
    logits = _tc_call(_tc_last_body, (npn, d_h), degs, a3, b3_p)
    return logits[:n, :d_out]


# speedup vs baseline: 1.2259x; 1.2242x over previous
import functools

import jax
import jax.numpy as jnp
from jax import lax
from jax.experimental import pallas as pl
from jax.experimental.pallas import tpu as pltpu
from jax.experimental.pallas import tpu_sc as plsc

NC = 2
NS = 16
NW = NC * NS
CHUNK = 128


def _mesh():
    return plsc.VectorSubcoreMesh(
        core_axis_name="c", subcore_axis_name="s",
        num_cores=NC, num_subcores=NS)


@functools.lru_cache(maxsize=None)
def _make_deg_kernel(npn: int, k: int):
    rows_per_tile = npn // NS

    @functools.partial(
        pl.kernel, mesh=_mesh(),
        out_type=jax.ShapeDtypeStruct((NC, 2, npn), jnp.float32),
        scratch_types=[
            pltpu.VMEM((k, CHUNK), jnp.int32),
            pltpu.VMEM((k, CHUNK), jnp.int32),
            pltpu.VMEM((CHUNK,), jnp.float32),
            pltpu.VMEM((CHUNK,), jnp.float32),
            pltpu.VMEM_SHARED((npn,), jnp.float32),
            pltpu.VMEM_SHARED((npn,), jnp.float32),
        ],
    )
    def deg_kernel(src_hbm, dst_hbm, out_hbm,
                   src_v, dst_v, ones_v, zeros_v, dego_s, degi_s):
        cid = lax.axis_index("c")
        sid = lax.axis_index("s")
        wid = sid * NC + cid

        def fill(i, _):
            ones_v[pl.ds(i * 16, 16)] = jnp.full((16,), 1.0, jnp.float32)
            zeros_v[pl.ds(i * 16, 16)] = jnp.zeros((16,), jnp.float32)
            return 0
        lax.fori_loop(0, CHUNK // 16, fill, 0)

        base = sid * rows_per_tile

        def zrow(i, _):
            pltpu.sync_copy(zeros_v, dego_s.at[pl.ds(base + i * CHUNK, CHUNK)])
            pltpu.sync_copy(zeros_v, degi_s.at[pl.ds(base + i * CHUNK, CHUNK)])
            return 0
        lax.fori_loop(0, rows_per_tile // CHUNK, zrow, 0)
        plsc.subcore_barrier()

        pltpu.sync_copy(src_hbm.at[wid], src_v)
        pltpu.sync_copy(dst_hbm.at[wid], dst_v)

        def body(j, _):
            pltpu.sync_copy(ones_v, dego_s.at[src_v.at[j]], add=True)
            pltpu.sync_copy(ones_v, degi_s.at[dst_v.at[j]], add=True)
            return 0
        lax.fori_loop(0, k, body, 0)
        plsc.subcore_barrier()

        pltpu.sync_copy(dego_s.at[pl.ds(base, rows_per_tile)],
                        out_hbm.at[cid, 0, pl.ds(base, rows_per_tile)])
        pltpu.sync_copy(degi_s.at[pl.ds(base, rows_per_tile)],
                        out_hbm.at[cid, 1, pl.ds(base, rows_per_tile)])

    return deg_kernel


@functools.lru_cache(maxsize=None)
def _make_agg_kernel(npn: int, d: int, k: int):
    rows_per_tile = npn // NS

    @functools.partial(
        pl.kernel, mesh=_mesh(),
        out_type=jax.ShapeDtypeStruct((NC, npn, d), jnp.float32),
        scratch_types=[
            pltpu.VMEM((k, CHUNK), jnp.int32),
            pltpu.VMEM((k, CHUNK), jnp.int32),
            pltpu.VMEM((CHUNK, d), jnp.float32),
            pltpu.VMEM_SHARED((npn, d), jnp.float32),
        ],
    )
    def agg_kernel(g_hbm, src_hbm, dst_hbm, out_hbm,
                   src_v, dst_v, rows_v, agg_s):
        cid = lax.axis_index("c")
        sid = lax.axis_index("s")
        wid = sid * NC + cid

        nv = CHUNK * d // 16

        def fz(i, _):
            rows_v[i // (d // 16), pl.ds((i % (d // 16)) * 16, 16)] = (
                jnp.zeros((16,), jnp.float32))
            return 0
        lax.fori_loop(0, nv, fz, 0)

        base = sid * rows_per_tile

        def zrow(i, _):
            pltpu.sync_copy(rows_v, agg_s.at[pl.ds(base + i * CHUNK, CHUNK)])
            return 0
        lax.fori_loop(0, rows_per_tile // CHUNK, zrow, 0)
        plsc.subcore_barrier()

        pltpu.sync_copy(src_hbm.at[wid], src_v)
        pltpu.sync_copy(dst_hbm.at[wid], dst_v)

        def body(j, _):
            pltpu.sync_copy(g_hbm.at[src_v.at[j]], rows_v)
            pltpu.sync_copy(rows_v, agg_s.at[dst_v.at[j]], add=True)
            return 0
        lax.fori_loop(0, k, body, 0)
        plsc.subcore_barrier()

        pltpu.sync_copy(agg_s.at[pl.ds(base, rows_per_tile)],
                        out_hbm.at[cid, pl.ds(base, rows_per_tile)])

    return agg_kernel


def _norm_cols(degs):
    norm_out = lax.rsqrt(jnp.maximum(degs[:, 0:1] + degs[:, 1:2], 1.0))
    norm_in = lax.rsqrt(jnp.maximum(degs[:, 2:3] + degs[:, 3:4], 1.0))
    return norm_out, norm_in


def _tc_first_body(degs_ref, x_ref, w_ref, g_ref):
    norm_out, _ = _norm_cols(degs_ref[...])
    xw = jnp.dot(x_ref[...], w_ref[...], preferred_element_type=jnp.float32)
    g_ref[...] = xw * norm_out


def _tc_mid_body(degs_ref, agg_ref, b_ref, w_ref, g_ref):
    norm_out, norm_in = _norm_cols(degs_ref[...])
    agg = agg_ref[0] + agg_ref[1]
    h = jnp.maximum(agg * norm_in + b_ref[...][None, :], 0.0)
    hw = jnp.dot(h, w_ref[...], preferred_element_type=jnp.float32)
    g_ref[...] = hw * norm_out


def _tc_last_body(degs_ref, agg_ref, b_ref, out_ref):
    _, norm_in = _norm_cols(degs_ref[...])
    agg = agg_ref[0] + agg_ref[1]
    out_ref[...] = agg * norm_in + b_ref[...][None, :]


def _tc_call(body, out_shape, *args):
    return pl.pallas_call(
        body, out_shape=jax.ShapeDtypeStruct(out_shape, jnp.float32))(*args)


def kernel(features, edge_index, W1, b1, W2, b2, W3, b3):
    n, d_in = features.shape
    e = edge_index.shape[1]
    d_h = W1.shape[1]
    d_out = W3.shape[1]

    k = -(-e // (NW * CHUNK))
    ep = NW * k * CHUNK
    npn = -(-(n + 1) // (NS * CHUNK)) * (NS * CHUNK)

    src = edge_index[0]
    dst = edge_index[1]
    pad = ep - e
    src_g = jnp.pad(src, (0, pad)).reshape(NW, k, CHUNK)
    dst_s = jnp.pad(dst, (0, pad), constant_values=n).reshape(NW, k, CHUNK)
    src_d = jnp.pad(src, (0, pad), constant_values=n).reshape(NW, k, CHUNK)

    x_p = jnp.pad(features, ((0, npn - n), (0, 0)))

    deg_parts = _make_deg_kernel(npn, k)(src_d, dst_s)
    degs = jnp.transpose(deg_parts, (2, 1, 0)).reshape(npn, 4)

    w3_p = jnp.pad(W3, ((0, 0), (0, d_h - d_out)))
    b3_p = jnp.pad(b3, (0, d_h - d_out))

    agg = _make_agg_kernel(npn, d_h, k)
    g1 = _tc_call(_tc_first_body, (npn, d_h), degs, x_p, W1)
    a1 = agg(g1, src_g, dst_s)
    g2 = _tc_call(_tc_mid_body, (npn, d_h), degs, a1, b1, W2)
    a2 = agg(g2, src_g, dst_s)
    g3 = _tc_call(_tc_mid_body, (npn, d_h), degs, a2, b2, w3_p)
    a3 = agg(g3, src_g, dst_s)
    logits = _tc_call(_tc_last_body, (npn, d_h), degs, a3, b3_p)
    return logits[:n, :d_out]


# speedup vs baseline: 1.5756x; 1.2852x over previous
import functools

import jax
import jax.numpy as jnp
from jax import lax
from jax.experimental import pallas as pl
from jax.experimental.pallas import tpu as pltpu
from jax.experimental.pallas import tpu_sc as plsc

NC = 2
NS = 16
NW = NC * NS
CHUNK = 128
FRAC0 = 0.64


def _mesh():
    return plsc.VectorSubcoreMesh(
        core_axis_name="c", subcore_axis_name="s",
        num_cores=NC, num_subcores=NS)


@functools.lru_cache(maxsize=None)
def _make_deg_kernel(npn: int, k: int):
    rows_per_tile = npn // NS

    @functools.partial(
        pl.kernel, mesh=_mesh(),
        out_type=jax.ShapeDtypeStruct((NC, 2, npn), jnp.float32),
        scratch_types=[
            pltpu.VMEM((k, CHUNK), jnp.int32),
            pltpu.VMEM((k, CHUNK), jnp.int32),
            pltpu.VMEM((CHUNK,), jnp.float32),
            pltpu.VMEM((CHUNK,), jnp.float32),
            pltpu.VMEM_SHARED((npn,), jnp.float32),
            pltpu.VMEM_SHARED((npn,), jnp.float32),
        ],
    )
    def deg_kernel(src_hbm, dst_hbm, out_hbm,
                   src_v, dst_v, ones_v, zeros_v, dego_s, degi_s):
        cid = lax.axis_index("c")
        sid = lax.axis_index("s")
        wid = sid * NC + cid

        def fill(i, _):
            ones_v[pl.ds(i * 16, 16)] = jnp.full((16,), 1.0, jnp.float32)
            zeros_v[pl.ds(i * 16, 16)] = jnp.zeros((16,), jnp.float32)
            return 0
        lax.fori_loop(0, CHUNK // 16, fill, 0)

        base = sid * rows_per_tile

        def zrow(i, _):
            pltpu.sync_copy(zeros_v, dego_s.at[pl.ds(base + i * CHUNK, CHUNK)])
            pltpu.sync_copy(zeros_v, degi_s.at[pl.ds(base + i * CHUNK, CHUNK)])
            return 0
        lax.fori_loop(0, rows_per_tile // CHUNK, zrow, 0)
        plsc.subcore_barrier()

        pltpu.sync_copy(src_hbm.at[wid], src_v)
        pltpu.sync_copy(dst_hbm.at[wid], dst_v)

        def body(j, _):
            pltpu.sync_copy(ones_v, dego_s.at[src_v.at[j]], add=True)
            pltpu.sync_copy(ones_v, degi_s.at[dst_v.at[j]], add=True)
            return 0
        lax.fori_loop(0, k, body, 0)
        plsc.subcore_barrier()

        pltpu.sync_copy(dego_s.at[pl.ds(base, rows_per_tile)],
                        out_hbm.at[cid, 0, pl.ds(base, rows_per_tile)])
        pltpu.sync_copy(degi_s.at[pl.ds(base, rows_per_tile)],
                        out_hbm.at[cid, 1, pl.ds(base, rows_per_tile)])

    return deg_kernel


@functools.lru_cache(maxsize=None)
def _make_agg_kernel(npn: int, d: int, k0: int, k1: int):
    rows_per_tile = npn // NS
    kbuf = max(k0, k1)
    kbuf += -kbuf % 8

    @functools.partial(
        pl.kernel, mesh=_mesh(),
        out_type=jax.ShapeDtypeStruct((NC, npn, d), jnp.float32),
        scratch_types=[
            pltpu.VMEM((kbuf, CHUNK), jnp.int32),
            pltpu.VMEM((kbuf, CHUNK), jnp.int32),
            pltpu.VMEM((CHUNK, d), jnp.float32),
            pltpu.VMEM_SHARED((npn, d), jnp.float32),
        ],
    )
    def agg_kernel(g_hbm, src_hbm, dst_hbm, out_hbm,
                   src_v, dst_v, rows_v, agg_s):
        cid = lax.axis_index("c")
        sid = lax.axis_index("s")

        nv = CHUNK * d // 16

        def fz(i, _):
            rows_v[i // (d // 16), pl.ds((i % (d // 16)) * 16, 16)] = (
                jnp.zeros((16,), jnp.float32))
            return 0
        lax.fori_loop(0, nv, fz, 0)

        base = sid * rows_per_tile

        def zrow(i, _):
            pltpu.sync_copy(rows_v, agg_s.at[pl.ds(base + i * CHUNK, CHUNK)])
            return 0
        lax.fori_loop(0, rows_per_tile // CHUNK, zrow, 0)
        plsc.subcore_barrier()

        cbase = cid * kbuf
        pltpu.sync_copy(src_hbm.at[sid, pl.ds(cbase, kbuf)], src_v)
        pltpu.sync_copy(dst_hbm.at[sid, pl.ds(cbase, kbuf)], dst_v)
        trip = jnp.where(cid == 0, k0, k1)

        def body(j, _):
            pltpu.sync_copy(g_hbm.at[src_v.at[j]], rows_v)
            pltpu.sync_copy(rows_v, agg_s.at[dst_v.at[j]], add=True)
            return 0
        lax.fori_loop(0, trip, body, 0)
        plsc.subcore_barrier()

        pltpu.sync_copy(agg_s.at[pl.ds(base, rows_per_tile)],
                        out_hbm.at[cid, pl.ds(base, rows_per_tile)])

    return agg_kernel


def _norm_cols(degs):
    norm_out = lax.rsqrt(jnp.maximum(degs[:, 0:1] + degs[:, 1:2], 1.0))
    norm_in = lax.rsqrt(jnp.maximum(degs[:, 2:3] + degs[:, 3:4], 1.0))
    return norm_out, norm_in


def _tc_first_body(degs_ref, x_ref, w_ref, g_ref):
    norm_out, _ = _norm_cols(degs_ref[...])
    xw = jnp.dot(x_ref[...], w_ref[...], preferred_element_type=jnp.float32)
    g_ref[...] = xw * norm_out


def _tc_mid_body(degs_ref, agg_ref, b_ref, w_ref, g_ref):
    norm_out, norm_in = _norm_cols(degs_ref[...])
    agg = agg_ref[0] + agg_ref[1]
    h = jnp.maximum(agg * norm_in + b_ref[...][None, :], 0.0)
    hw = jnp.dot(h, w_ref[...], preferred_element_type=jnp.float32)
    g_ref[...] = hw * norm_out


def _tc_last_body(degs_ref, agg_ref, b_ref, out_ref):
    _, norm_in = _norm_cols(degs_ref[...])
    agg = agg_ref[0] + agg_ref[1]
    out_ref[...] = agg * norm_in + b_ref[...][None, :]


def _tc_call(body, out_shape, *args):
    return pl.pallas_call(
        body, out_shape=jax.ShapeDtypeStruct(out_shape, jnp.float32))(*args)


def kernel(features, edge_index, W1, b1, W2, b2, W3, b3):
    n, d_in = features.shape
    e = edge_index.shape[1]
    d_h = W1.shape[1]
    d_out = W3.shape[1]

    k = -(-e // (NW * CHUNK))
    ep = NW * k * CHUNK
    npn = -(-(n + 1) // (NS * CHUNK)) * (NS * CHUNK)

    src = edge_index[0]
    dst = edge_index[1]
    pad = ep - e
    dst_s = jnp.pad(dst, (0, pad), constant_values=n).reshape(NW, k, CHUNK)
    src_d = jnp.pad(src, (0, pad), constant_values=n).reshape(NW, k, CHUNK)

    e_pair = -(-e // NS)
    kp = -(-e_pair // CHUNK)
    k0 = max(1, min(kp - 1, round(kp * FRAC0)))
    k1 = kp - k0
    kbuf = max(k0, k1)
    kbuf += -kbuf % 8

    def _pair_layout(v, fill):
        vp = jnp.pad(v, (0, NS * e_pair - e), constant_values=fill)
        vp = vp.reshape(NS, e_pair)
        c0 = k0 * CHUNK
        p0 = jnp.pad(vp[:, :c0], ((0, 0), (0, kbuf * CHUNK - c0)),
                     constant_values=fill)
        p1 = jnp.pad(vp[:, c0:], ((0, 0), (0, kbuf * CHUNK - (e_pair - c0))),
                     constant_values=fill)
        return jnp.concatenate([p0, p1], axis=1).reshape(NS, 2 * kbuf, CHUNK)

    src_g = _pair_layout(src, 0)
    dst_g = _pair_layout(dst, n)

    x_p = jnp.pad(features, ((0, npn - n), (0, 0)))

    deg_parts = _make_deg_kernel(npn, k)(src_d, dst_s)
    degs = jnp.transpose(deg_parts, (2, 1, 0)).reshape(npn, 4)

    w3_p = jnp.pad(W3, ((0, 0), (0, d_h - d_out)))
    b3_p = jnp.pad(b3, (0, d_h - d_out))

    agg = _make_agg_kernel(npn, d_h, k0, k1)
    g1 = _tc_call(_tc_first_body, (npn, d_h), degs, x_p, W1)
    a1 = agg(g1, src_g, dst_g)
    g2 = _tc_call(_tc_mid_body, (npn, d_h), degs, a1, b1, W2)
    a2 = agg(g2, src_g, dst_g)
    g3 = _tc_call(_tc_mid_body, (npn, d_h), degs, a2, b2, w3_p)
    a3 = agg(g3, src_g, dst_g)
    logits = _tc_call(_tc_last_body, (npn, d_h), degs, a3, b3_p)
    return logits[:n, :d_out]


# speedup vs baseline: 1.6821x; 1.0676x over previous
import functools

import jax
import jax.numpy as jnp
from jax import lax
from jax.experimental import pallas as pl
from jax.experimental.pallas import tpu as pltpu
from jax.experimental.pallas import tpu_sc as plsc

NC = 2
NS = 16
NW = NC * NS
CHUNK = 128
FRAC0 = 0.58


def _mesh():
    return plsc.VectorSubcoreMesh(
        core_axis_name="c", subcore_axis_name="s",
        num_cores=NC, num_subcores=NS)


@functools.lru_cache(maxsize=None)
def _make_deg_kernel(npn: int, k: int):
    rows_per_tile = npn // NS

    @functools.partial(
        pl.kernel, mesh=_mesh(),
        out_type=jax.ShapeDtypeStruct((NC, 2, npn), jnp.float32),
        scratch_types=[
            pltpu.VMEM((k, CHUNK), jnp.int32),
            pltpu.VMEM((k, CHUNK), jnp.int32),
            pltpu.VMEM((CHUNK,), jnp.float32),
            pltpu.VMEM((CHUNK,), jnp.float32),
            pltpu.VMEM_SHARED((npn,), jnp.float32),
            pltpu.VMEM_SHARED((npn,), jnp.float32),
        ],
    )
    def deg_kernel(src_hbm, dst_hbm, out_hbm,
                   src_v, dst_v, ones_v, zeros_v, dego_s, degi_s):
        cid = lax.axis_index("c")
        sid = lax.axis_index("s")
        wid = sid * NC + cid

        def fill(i, _):
            ones_v[pl.ds(i * 16, 16)] = jnp.full((16,), 1.0, jnp.float32)
            zeros_v[pl.ds(i * 16, 16)] = jnp.zeros((16,), jnp.float32)
            return 0
        lax.fori_loop(0, CHUNK // 16, fill, 0)

        base = sid * rows_per_tile

        def zrow(i, _):
            pltpu.sync_copy(zeros_v, dego_s.at[pl.ds(base + i * CHUNK, CHUNK)])
            pltpu.sync_copy(zeros_v, degi_s.at[pl.ds(base + i * CHUNK, CHUNK)])
            return 0
        lax.fori_loop(0, rows_per_tile // CHUNK, zrow, 0)
        plsc.subcore_barrier()

        pltpu.sync_copy(src_hbm.at[wid], src_v)
        pltpu.sync_copy(dst_hbm.at[wid], dst_v)

        def body(j, _):
            pltpu.sync_copy(ones_v, dego_s.at[src_v.at[j]], add=True)
            pltpu.sync_copy(ones_v, degi_s.at[dst_v.at[j]], add=True)
            return 0
        lax.fori_loop(0, k, body, 0)
        plsc.subcore_barrier()

        pltpu.sync_copy(dego_s.at[pl.ds(base, rows_per_tile)],
                        out_hbm.at[cid, 0, pl.ds(base, rows_per_tile)])
        pltpu.sync_copy(degi_s.at[pl.ds(base, rows_per_tile)],
                        out_hbm.at[cid, 1, pl.ds(base, rows_per_tile)])

    return deg_kernel


@functools.lru_cache(maxsize=None)
def _make_agg_kernel(npn: int, d: int, k0: int, k1: int):
    rows_per_tile = npn // NS
    kbuf = max(k0, k1)
    kbuf += -kbuf % 8

    @functools.partial(
        pl.kernel, mesh=_mesh(),
        out_type=jax.ShapeDtypeStruct((NC, npn, d), jnp.float32),
        scratch_types=[
            pltpu.VMEM((kbuf, CHUNK), jnp.int32),
            pltpu.VMEM((kbuf, CHUNK), jnp.int32),
            pltpu.VMEM((CHUNK, d), jnp.float32),
            pltpu.VMEM_SHARED((npn, d), jnp.float32),
        ],
    )
    def agg_kernel(g_hbm, src_hbm, dst_hbm, out_hbm,
                   src_v, dst_v, rows_v, agg_s):
        cid = lax.axis_index("c")
        sid = lax.axis_index("s")

        nv = CHUNK * d // 16

        def fz(i, _):
            rows_v[i // (d // 16), pl.ds((i % (d // 16)) * 16, 16)] = (
                jnp.zeros((16,), jnp.float32))
            return 0
        lax.fori_loop(0, nv, fz, 0)

        base = sid * rows_per_tile

        def zrow(i, _):
            pltpu.sync_copy(rows_v, agg_s.at[pl.ds(base + i * CHUNK, CHUNK)])
            return 0
        lax.fori_loop(0, rows_per_tile // CHUNK, zrow, 0)
        plsc.subcore_barrier()

        cbase = cid * kbuf
        pltpu.sync_copy(src_hbm.at[sid, pl.ds(cbase, kbuf)], src_v)
        pltpu.sync_copy(dst_hbm.at[sid, pl.ds(cbase, kbuf)], dst_v)
        trip = jnp.where(cid == 0, k0, k1)

        def body(j, _):
            pltpu.sync_copy(g_hbm.at[src_v.at[j]], rows_v)
            pltpu.sync_copy(rows_v, agg_s.at[dst_v.at[j]], add=True)
            return 0
        lax.fori_loop(0, trip, body, 0)
        plsc.subcore_barrier()

        pltpu.sync_copy(agg_s.at[pl.ds(base, rows_per_tile)],
                        out_hbm.at[cid, pl.ds(base, rows_per_tile)])

    return agg_kernel


def _norm_cols(degs):
    norm_out = lax.rsqrt(jnp.maximum(degs[:, 0:1] + degs[:, 1:2], 1.0))
    norm_in = lax.rsqrt(jnp.maximum(degs[:, 2:3] + degs[:, 3:4], 1.0))
    return norm_out, norm_in


def _tc_first_body(degs_ref, x_ref, w_ref, g_ref):
    norm_out, _ = _norm_cols(degs_ref[...])
    xw = jnp.dot(x_ref[...], w_ref[...], preferred_element_type=jnp.float32)
    g_ref[...] = xw * norm_out


def _tc_mid_body(degs_ref, agg_ref, b_ref, w_ref, g_ref):
    norm_out, norm_in = _norm_cols(degs_ref[...])
    agg = agg_ref[0] + agg_ref[1]
    h = jnp.maximum(agg * norm_in + b_ref[...][None, :], 0.0)
    hw = jnp.dot(h, w_ref[...], preferred_element_type=jnp.float32)
    g_ref[...] = hw * norm_out


def _tc_last_body(degs_ref, agg_ref, b_ref, out_ref):
    _, norm_in = _norm_cols(degs_ref[...])
    agg = agg_ref[0] + agg_ref[1]
    out_ref[...] = agg * norm_in + b_ref[...][None, :]


def _tc_call(body, out_shape, *args):
    return pl.pallas_call(
        body, out_shape=jax.ShapeDtypeStruct(out_shape, jnp.float32))(*args)


def kernel(features, edge_index, W1, b1, W2, b2, W3, b3):
    n, d_in = features.shape
    e = edge_index.shape[1]
    d_h = W1.shape[1]
    d_out = W3.shape[1]

    k = -(-e // (NW * CHUNK))
    ep = NW * k * CHUNK
    npn = -(-(n + 1) // (NS * CHUNK)) * (NS * CHUNK)

    src = edge_index[0]
    dst = edge_index[1]
    pad = ep - e
    dst_s = jnp.pad(dst, (0, pad), constant_values=n).reshape(NW, k, CHUNK)
    src_d = jnp.pad(src, (0, pad), constant_values=n).reshape(NW, k, CHUNK)

    e_pair = -(-e // NS)
    kp = -(-e_pair // CHUNK)
    k0 = max(1, min(kp - 1, round(kp * FRAC0)))
    k1 = kp - k0
    kbuf = max(k0, k1)
    kbuf += -kbuf % 8

    def _pair_layout(v, fill):
        vp = jnp.pad(v, (0, NS * e_pair - e), constant_values=fill)
        vp = vp.reshape(NS, e_pair)
        c0 = k0 * CHUNK
        p0 = jnp.pad(vp[:, :c0], ((0, 0), (0, kbuf * CHUNK - c0)),
                     constant_values=fill)
        p1 = jnp.pad(vp[:, c0:], ((0, 0), (0, kbuf * CHUNK - (e_pair - c0))),
                     constant_values=fill)
        return jnp.concatenate([p0, p1], axis=1).reshape(NS, 2 * kbuf, CHUNK)

    src_g = _pair_layout(src, 0)
    dst_g = _pair_layout(dst, n)

    x_p = jnp.pad(features, ((0, npn - n), (0, 0)))

    deg_parts = _make_deg_kernel(npn, k)(src_d, dst_s)
    degs = jnp.transpose(deg_parts, (2, 1, 0)).reshape(npn, 4)

    w3_p = jnp.pad(W3, ((0, 0), (0, d_h - d_out)))
    b3_p = jnp.pad(b3, (0, d_h - d_out))

    agg = _make_agg_kernel(npn, d_h, k0, k1)
    g1 = _tc_call(_tc_first_body, (npn, d_h), degs, x_p, W1)
    a1 = agg(g1, src_g, dst_g)
    g2 = _tc_call(_tc_mid_body, (npn, d_h), degs, a1, b1, W2)
    a2 = agg(g2, src_g, dst_g)
    g3 = _tc_call(_tc_mid_body, (npn, d_h), degs, a2, b2, w3_p)
    a3 = agg(g3, src_g, dst_g)
    logits = _tc_call(_tc_last_body, (npn, d_h), degs, a3, b3_p)
    return logits[:n, :d_out]


# speedup vs baseline: 1.7096x; 1.0164x over previous
import functools

import jax
import jax.numpy as jnp
from jax import lax
from jax.experimental import pallas as pl
from jax.experimental.pallas import tpu as pltpu
from jax.experimental.pallas import tpu_sc as plsc

NC = 2
NS = 16
NW = NC * NS
CHUNK = 128
FRAC0 = 0.555


def _mesh():
    return plsc.VectorSubcoreMesh(
        core_axis_name="c", subcore_axis_name="s",
        num_cores=NC, num_subcores=NS)


@functools.lru_cache(maxsize=None)
def _make_deg_kernel(npn: int, k: int):
    rows_per_tile = npn // NS

    @functools.partial(
        pl.kernel, mesh=_mesh(),
        out_type=jax.ShapeDtypeStruct((NC, 2, npn), jnp.float32),
        scratch_types=[
            pltpu.VMEM((k, CHUNK), jnp.int32),
            pltpu.VMEM((k, CHUNK), jnp.int32),
            pltpu.VMEM((CHUNK,), jnp.float32),
            pltpu.VMEM((CHUNK,), jnp.float32),
            pltpu.VMEM_SHARED((npn,), jnp.float32),
            pltpu.VMEM_SHARED((npn,), jnp.float32),
        ],
    )
    def deg_kernel(src_hbm, dst_hbm, out_hbm,
                   src_v, dst_v, ones_v, zeros_v, dego_s, degi_s):
        cid = lax.axis_index("c")
        sid = lax.axis_index("s")
        wid = sid * NC + cid

        def fill(i, _):
            ones_v[pl.ds(i * 16, 16)] = jnp.full((16,), 1.0, jnp.float32)
            zeros_v[pl.ds(i * 16, 16)] = jnp.zeros((16,), jnp.float32)
            return 0
        lax.fori_loop(0, CHUNK // 16, fill, 0)

        base = sid * rows_per_tile

        def zrow(i, _):
            pltpu.sync_copy(zeros_v, dego_s.at[pl.ds(base + i * CHUNK, CHUNK)])
            pltpu.sync_copy(zeros_v, degi_s.at[pl.ds(base + i * CHUNK, CHUNK)])
            return 0
        lax.fori_loop(0, rows_per_tile // CHUNK, zrow, 0)
        plsc.subcore_barrier()

        pltpu.sync_copy(src_hbm.at[wid], src_v)
        pltpu.sync_copy(dst_hbm.at[wid], dst_v)

        def body(j, _):
            pltpu.sync_copy(ones_v, dego_s.at[src_v.at[j]], add=True)
            pltpu.sync_copy(ones_v, degi_s.at[dst_v.at[j]], add=True)
            return 0
        lax.fori_loop(0, k, body, 0)
        plsc.subcore_barrier()

        pltpu.sync_copy(dego_s.at[pl.ds(base, rows_per_tile)],
                        out_hbm.at[cid, 0, pl.ds(base, rows_per_tile)])
        pltpu.sync_copy(degi_s.at[pl.ds(base, rows_per_tile)],
                        out_hbm.at[cid, 1, pl.ds(base, rows_per_tile)])

    return deg_kernel


@functools.lru_cache(maxsize=None)
def _make_agg_kernel(npn: int, d: int, k0: int, k1: int):
    rows_per_tile = npn // NS
    kbuf = max(k0, k1)
    kbuf += -kbuf % 8

    @functools.partial(
        pl.kernel, mesh=_mesh(),
        out_type=jax.ShapeDtypeStruct((NC, npn, d), jnp.float32),
        scratch_types=[
            pltpu.VMEM((kbuf, CHUNK), jnp.int32),
            pltpu.VMEM((kbuf, CHUNK), jnp.int32),
            pltpu.VMEM((CHUNK, d), jnp.float32),
            pltpu.VMEM_SHARED((npn, d), jnp.float32),
        ],
    )
    def agg_kernel(g_hbm, src_hbm, dst_hbm, out_hbm,
                   src_v, dst_v, rows_v, agg_s):
        cid = lax.axis_index("c")
        sid = lax.axis_index("s")

        nv = CHUNK * d // 16

        def fz(i, _):
            rows_v[i // (d // 16), pl.ds((i % (d // 16)) * 16, 16)] = (
                jnp.zeros((16,), jnp.float32))
            return 0
        lax.fori_loop(0, nv, fz, 0)

        base = sid * rows_per_tile

        def zrow(i, _):
            pltpu.sync_copy(rows_v, agg_s.at[pl.ds(base + i * CHUNK, CHUNK)])
            return 0
        lax.fori_loop(0, rows_per_tile // CHUNK, zrow, 0)
        plsc.subcore_barrier()

        cbase = cid * kbuf
        pltpu.sync_copy(src_hbm.at[sid, pl.ds(cbase, kbuf)], src_v)
        pltpu.sync_copy(dst_hbm.at[sid, pl.ds(cbase, kbuf)], dst_v)
        trip = jnp.where(cid == 0, k0, k1)

        def body(j, _):
            pltpu.sync_copy(g_hbm.at[src_v.at[j]], rows_v)
            pltpu.sync_copy(rows_v, agg_s.at[dst_v.at[j]], add=True)
            return 0
        lax.fori_loop(0, trip, body, 0)
        plsc.subcore_barrier()

        pltpu.sync_copy(agg_s.at[pl.ds(base, rows_per_tile)],
                        out_hbm.at[cid, pl.ds(base, rows_per_tile)])

    return agg_kernel


def _norm_cols(degs):
    norm_out = lax.rsqrt(jnp.maximum(degs[:, 0:1] + degs[:, 1:2], 1.0))
    norm_in = lax.rsqrt(jnp.maximum(degs[:, 2:3] + degs[:, 3:4], 1.0))
    return norm_out, norm_in


def _tc_first_body(degs_ref, x_ref, w_ref, g_ref):
    norm_out, _ = _norm_cols(degs_ref[...])
    xw = jnp.dot(x_ref[...], w_ref[...], preferred_element_type=jnp.float32)
    g_ref[...] = xw * norm_out


def _tc_mid_body(degs_ref, agg_ref, b_ref, w_ref, g_ref):
    norm_out, norm_in = _norm_cols(degs_ref[...])
    agg = agg_ref[0] + agg_ref[1]
    h = jnp.maximum(agg * norm_in + b_ref[...][None, :], 0.0)
    hw = jnp.dot(h, w_ref[...], preferred_element_type=jnp.float32)
    g_ref[...] = hw * norm_out


def _tc_last_body(degs_ref, agg_ref, b_ref, out_ref):
    _, norm_in = _norm_cols(degs_ref[...])
    agg = agg_ref[0] + agg_ref[1]
    out_ref[...] = agg * norm_in + b_ref[...][None, :]


def _tc_call(body, out_shape, *args):
    return pl.pallas_call(
        body, out_shape=jax.ShapeDtypeStruct(out_shape, jnp.float32))(*args)


def kernel(features, edge_index, W1, b1, W2, b2, W3, b3):
    n, d_in = features.shape
    e = edge_index.shape[1]
    d_h = W1.shape[1]
    d_out = W3.shape[1]

    k = -(-e // (NW * CHUNK))
    ep = NW * k * CHUNK
    npn = -(-(n + 1) // (NS * CHUNK)) * (NS * CHUNK)

    src = edge_index[0]
    dst = edge_index[1]
    pad = ep - e
    dst_s = jnp.pad(dst, (0, pad), constant_values=n).reshape(NW, k, CHUNK)
    src_d = jnp.pad(src, (0, pad), constant_values=n).reshape(NW, k, CHUNK)

    e_pair = -(-e // NS)
    kp = -(-e_pair // CHUNK)
    k0 = max(1, min(kp - 1, round(kp * FRAC0)))
    k1 = kp - k0
    kbuf = max(k0, k1)
    kbuf += -kbuf % 8

    def _pair_layout(v, fill):
        vp = jnp.pad(v, (0, NS * e_pair - e), constant_values=fill)
        vp = vp.reshape(NS, e_pair)
        c0 = k0 * CHUNK
        p0 = jnp.pad(vp[:, :c0], ((0, 0), (0, kbuf * CHUNK - c0)),
                     constant_values=fill)
        p1 = jnp.pad(vp[:, c0:], ((0, 0), (0, kbuf * CHUNK - (e_pair - c0))),
                     constant_values=fill)
        return jnp.concatenate([p0, p1], axis=1).reshape(NS, 2 * kbuf, CHUNK)

    src_g = _pair_layout(src, 0)
    dst_g = _pair_layout(dst, n)

    x_p = jnp.pad(features, ((0, npn - n), (0, 0)))

    deg_parts = _make_deg_kernel(npn, k)(src_d, dst_s)
    degs = jnp.transpose(deg_parts, (2, 1, 0)).reshape(npn, 4)

    w3_p = jnp.pad(W3, ((0, 0), (0, d_h - d_out)))
    b3_p = jnp.pad(b3, (0, d_h - d_out))

    agg = _make_agg_kernel(npn, d_h, k0, k1)
    g1 = _tc_call(_tc_first_body, (npn, d_h), degs, x_p, W1)
    a1 = agg(g1, src_g, dst_g)
    g2 = _tc_call(_tc_mid_body, (npn, d_h), degs, a1, b1, W2)
    a2 = agg(g2, src_g, dst_g)
    g3 = _tc_call(_tc_mid_body, (npn, d_h), degs, a2, b2, w3_p)
    a3 = agg(g3, src_g, dst_g)
    logits = _tc_call(_tc_last_body, (npn, d_h), degs, a3, b3_p)
    return logits[:n, :d_out]
